# Initial kernel scaffold; baseline (speedup 1.0000x reference)
import jax
import jax.numpy as jnp
from jax.experimental import pallas as pl


def kernel(x, Adj_, W1, b1, W2, b2, P1, pb1, P2, pb2):
    raise NotImplementedError("write your pallas kernel here")


# speedup vs baseline: 1.0212x; 1.0212x over previous
import jax
import jax.numpy as jnp
from jax.experimental import pallas as pl

N = 10000
D = 128
BM = 400
GRID = N // BM


def _h0_body(x_ref, w1_ref, b1_ref, out_ref):
    out_ref[...] = (
        jnp.dot(x_ref[...], w1_ref[...], preferred_element_type=jnp.float32)
        + b1_ref[...]
    )


def _pass_a_body(adj_ref, h0_ref, w2_ref, b2_ref, out_ref):
    h1 = jnp.dot(adj_ref[...], h0_ref[...], preferred_element_type=jnp.float32)
    h1 = jnp.maximum(h1, 0.0)
    out_ref[...] = (
        jnp.dot(h1, w2_ref[...], preferred_element_type=jnp.float32) + b2_ref[...]
    )


def _pass_b_body(adj_ref, h2_ref, p1_ref, pb1_ref, p2_ref, pb2_ref,
                 z_ref, xout_ref):
    xo = jnp.dot(adj_ref[...], h2_ref[...], preferred_element_type=jnp.float32)
    xout_ref[...] = xo
    t = jnp.maximum(
        jnp.dot(xo, p1_ref[...], preferred_element_type=jnp.float32)
        + pb1_ref[...], 0.0)
    z_ref[...] = (
        jnp.dot(t, p2_ref[...], preferred_element_type=jnp.float32) + pb2_ref[...]
    )


def kernel(x, Adj_, W1, b1, W2, b2, P1, pb1, P2, pb2):
    f32 = jnp.float32
    b1r = b1.reshape(1, D)
    b2r = b2.reshape(1, D)
    pb1r = pb1.reshape(1, D)
    pb2r = pb2.reshape(1, D)

    full = lambda r, c: pl.BlockSpec((r, c), lambda i: (0, 0))

    h0 = pl.pallas_call(
        _h0_body,
        grid=(1,),
        in_specs=[full(N, D), full(D, D), full(1, D)],
        out_specs=pl.BlockSpec((N, D), lambda i: (0, 0)),
        out_shape=jax.ShapeDtypeStruct((N, D), f32),
    )(x, W1, b1r)

    h2 = pl.pallas_call(
        _pass_a_body,
        grid=(GRID,),
        in_specs=[
            pl.BlockSpec((BM, N), lambda i: (i, 0)),
            full(N, D), full(D, D), full(1, D),
        ],
        out_specs=pl.BlockSpec((BM, D), lambda i: (i, 0)),
        out_shape=jax.ShapeDtypeStruct((N, D), f32),
    )(Adj_, h0, W2, b2r)

    z, x_out = pl.pallas_call(
        _pass_b_body,
        grid=(GRID,),
        in_specs=[
            pl.BlockSpec((BM, N), lambda i: (i, 0)),
            full(N, D), full(D, D), full(1, D), full(D, D), full(1, D),
        ],
        out_specs=[
            pl.BlockSpec((BM, D), lambda i: (i, 0)),
            pl.BlockSpec((BM, D), lambda i: (i, 0)),
        ],
        out_shape=[
            jax.ShapeDtypeStruct((N, D), f32),
            jax.ShapeDtypeStruct((N, D), f32),
        ],
    )(Adj_, h2, P1, pb1r, P2, pb2r)

    return (z, x_out)
